# baseline (speedup 1.0000x reference)
import functools

import jax
import jax.numpy as jnp
from jax.experimental import pallas as pl
from jax.experimental.pallas import tpu as pltpu

B, L, D = 1024, 200, 128
BB = 32


def _append_emb_kernel(lbl_ref, x_ref, emb_ref, out_ref):
    i = pl.program_id(0)
    out_ref[:, :, :D] = x_ref[...]
    for j in range(BB):
        lbl = lbl_ref[i * BB + j]
        row = emb_ref[lbl, :]
        out_ref[j, :, D:] = jnp.broadcast_to(row[None, :], (L, D))


@jax.jit
def kernel(x, labels_pointer, emb_table):
    grid = (B // BB,)
    grid_spec = pltpu.PrefetchScalarGridSpec(
        num_scalar_prefetch=1,
        grid=grid,
        in_specs=[
            pl.BlockSpec((BB, L, D), lambda i, lbl: (i, 0, 0)),
            pl.BlockSpec(emb_table.shape, lambda i, lbl: (0, 0)),
        ],
        out_specs=pl.BlockSpec((BB, L, 2 * D), lambda i, lbl: (i, 0, 0)),
    )
    return pl.pallas_call(
        _append_emb_kernel,
        grid_spec=grid_spec,
        out_shape=jax.ShapeDtypeStruct((B, L, 2 * D), x.dtype),
        compiler_params=pltpu.CompilerParams(
            dimension_semantics=("parallel",)),
    )(labels_pointer, x, emb_table)
